# baseline (speedup 1.0000x reference)
import functools

import jax
import jax.numpy as jnp
from jax import lax
from jax.experimental import pallas as pl
from jax.experimental.pallas import tpu as pltpu
from jax.experimental.pallas import tpu_sc as plsc

_NC = 2
_NS = 16
_NSP = 10
_NH = _NS - _NSP
_STAGE_ROUND = 10_000
_Y = 27_880
_X = 21_800


def _sc_gather(idx_flat, values):
    total = idx_flat.shape[0]
    nvals = values.shape[0]
    assert total == 2 * (_NSP * _Y + _NH * _X)
    assert _Y % 8 == 0 and _X % 8 == 0
    stage_per = nvals // _NSP
    assert stage_per % _STAGE_ROUND == 0 and _STAGE_ROUND % 8 == 0
    nrounds = stage_per // _STAGE_ROUND
    hbase = 2 * _NSP * _Y
    assert hbase % 8 == 0
    mesh = plsc.VectorSubcoreMesh(core_axis_name="c", subcore_axis_name="s")

    @functools.partial(
        pl.kernel,
        mesh=mesh,
        out_type=jax.ShapeDtypeStruct((total,), jnp.float32),
        scratch_types=[
            pltpu.VMEM_SHARED((nvals,), jnp.float32),
            pltpu.VMEM((_Y,), jnp.int32),
            pltpu.VMEM((_Y,), jnp.float32),
            pltpu.SemaphoreType.DMA,
            pltpu.SemaphoreType.DMA,
            pltpu.SemaphoreType.DMA,
        ],
    )
    def k(idx_hbm, values_hbm, out_hbm, shared, idx_v, rows_v, sem, isem,
          stsem):
        c = lax.axis_index("c")
        s = lax.axis_index("s")
        is_sp = s < _NSP
        base = jnp.where(is_sp, (s * _NC + c) * _Y,
                         hbase + ((s - _NSP) * _NC + c) * _X)

        @pl.when(jnp.logical_not(is_sp))
        def _hbm_path():
            pltpu.sync_copy(idx_hbm.at[pl.ds(base, _X)],
                            idx_v.at[pl.ds(0, _X)])
            cp = pltpu.async_copy(values_hbm.at[idx_v.at[pl.ds(0, _X)]],
                                  rows_v.at[pl.ds(0, _X)], sem)
            plsc.subcore_barrier()
            cp.wait()
            pltpu.sync_copy(rows_v.at[pl.ds(0, _X)],
                            out_hbm.at[pl.ds(base, _X)])

        @pl.when(is_sp)
        def _spmem_path():
            idx_cp = pltpu.async_copy(idx_hbm.at[pl.ds(base, _Y)], idx_v,
                                      isem)
            r = _STAGE_ROUND
            bufs = [rows_v.at[pl.ds(0, r)], rows_v.at[pl.ds(r, r)]]
            my0 = s * stage_per
            ins = [None] * nrounds
            outs = [None] * nrounds
            ins[0] = pltpu.async_copy(values_hbm.at[pl.ds(my0, r)],
                                      bufs[0], stsem)
            for j in range(nrounds):
                ins[j].wait()
                outs[j] = pltpu.async_copy(
                    bufs[j % 2], shared.at[pl.ds(my0 + j * r, r)], sem)
                if j + 1 < nrounds:
                    if j >= 1:
                        outs[j - 1].wait()
                    ins[j + 1] = pltpu.async_copy(
                        values_hbm.at[pl.ds(my0 + (j + 1) * r, r)],
                        bufs[(j + 1) % 2], stsem)
            outs[nrounds - 2].wait()
            outs[nrounds - 1].wait()
            plsc.subcore_barrier()
            idx_cp.wait()
            pltpu.async_copy(shared.at[idx_v], rows_v, sem).wait()
            pltpu.sync_copy(rows_v, out_hbm.at[pl.ds(base, _Y)])

    return k(idx_flat, values)


def kernel(indices, values):
    b, l = indices.shape
    idx_flat = indices.T.reshape(-1).astype(jnp.int32)
    out = _sc_gather(idx_flat, values)
    return out.reshape(l, b, 1).transpose(1, 0, 2)


# speedup vs baseline: 1.1994x; 1.1994x over previous
import functools

import jax
import jax.numpy as jnp
from jax import lax
from jax.experimental import pallas as pl
from jax.experimental.pallas import tpu as pltpu
from jax.experimental.pallas import tpu_sc as plsc

_NC = 2
_NS = 16
_NW = _NC * _NS
_STAGERS = 10
_STAGE_ROUND = 25_000


def _sc_gather(idx_flat, values):
    total = idx_flat.shape[0]
    nvals = values.shape[0]
    assert total % (8 * _NW) == 0
    per_w = total // _NW
    stage_per = nvals // _STAGERS
    assert stage_per % _STAGE_ROUND == 0 and _STAGE_ROUND % 8 == 0
    assert _STAGE_ROUND <= per_w
    nrounds = stage_per // _STAGE_ROUND
    mesh = plsc.VectorSubcoreMesh(core_axis_name="c", subcore_axis_name="s")

    @functools.partial(
        pl.kernel,
        mesh=mesh,
        out_type=jax.ShapeDtypeStruct((total,), jnp.float32),
        scratch_types=[
            pltpu.VMEM_SHARED((nvals,), jnp.float32),
            pltpu.VMEM((per_w,), jnp.int32),
            pltpu.VMEM((per_w,), jnp.float32),
            pltpu.SemaphoreType.DMA,
            pltpu.SemaphoreType.DMA,
        ],
    )
    def k(idx_hbm, values_hbm, out_hbm, shared, idx_v, rows_v, sem, isem):
        c = lax.axis_index("c")
        s = lax.axis_index("s")
        wid = s * _NC + c
        base = wid * per_w
        idx_cp = pltpu.async_copy(idx_hbm.at[pl.ds(base, per_w)], idx_v, isem)

        @pl.when(s < _STAGERS)
        def _stage():
            bounce = rows_v.at[pl.ds(0, _STAGE_ROUND)]
            for j in range(nrounds):
                off = s * stage_per + j * _STAGE_ROUND
                pltpu.sync_copy(values_hbm.at[pl.ds(off, _STAGE_ROUND)],
                                bounce)
                pltpu.sync_copy(bounce, shared.at[pl.ds(off, _STAGE_ROUND)])

        plsc.subcore_barrier()
        idx_cp.wait()
        pltpu.async_copy(shared.at[idx_v], rows_v, sem).wait()
        pltpu.sync_copy(rows_v, out_hbm.at[pl.ds(base, per_w)])

    return k(idx_flat, values)


def kernel(indices, values):
    b, l = indices.shape
    idx_flat = indices.T.reshape(-1).astype(jnp.int32)
    out = _sc_gather(idx_flat, values)
    return out.reshape(l, b, 1).transpose(1, 0, 2)
